# baseline (speedup 1.0000x reference)
import jax
import jax.numpy as jnp
from jax import lax
from jax.experimental import pallas as pl
from jax.experimental.pallas import tpu as pltpu
from jax.experimental.pallas import tpu_sc as plsc

B = 16384
S = 64
NC = 2
NS = 16
NW = NC * NS
L = 16
PB = B // NW
N_TABLE = 100000

_TBLK = 16384
_TGRID = (N_TABLE + _TBLK - 1) // _TBLK
_SBLK = 4096


def _sc_body(m_hbm, s2_hbm, wu_hbm, b_hbm, ow_hbm,
             idx_v, wu_v, s_v, bm_v, sem_wu, sem_b, sem_s):
    wid = lax.axis_index("s") * NC + lax.axis_index("c")
    base = wid * PB
    half = base // (B // 2)

    pltpu.sync_copy(m_hbm.at[pl.ds(base, PB)], idx_v)
    cwu = pltpu.make_async_copy(wu_hbm.at[idx_v], wu_v, sem_wu)
    cb = pltpu.make_async_copy(b_hbm.at[idx_v], bm_v.at[pl.ds(0, PB)], sem_b)
    cs = pltpu.make_async_copy(
        s2_hbm.at[pl.ds(base % (B // 2), PB), pl.ds(half * S, S)], s_v, sem_s)
    cwu.start()
    cb.start()
    cs.start()
    cwu.wait()
    cs.wait()
    cb.wait()

    lane15 = jnp.full((L,), 15, jnp.int32)
    lane0 = jnp.zeros((L,), jnp.int32)

    @plsc.parallel_loop(0, PB, 1, unroll=8)
    def _body(i):
        sv = [s_v[i, pl.ds(16 * k, L)] for k in range(S // L)]
        wv = [wu_v[i, pl.ds(16 * k, L)] for k in range(S // L)]
        uv = [wu_v[i, pl.ds(S + 16 * k, L)] for k in range(S // L)]
        p = (wv[0] * sv[0] + wv[1] * sv[1]) + (wv[2] * sv[2] + wv[3] * sv[3])
        c = plsc.cumsum(p)
        inner = jnp.take_along_axis(c, lane15, axis=0)
        bvec = jnp.take_along_axis(bm_v[pl.ds(i, L)], lane0, axis=0)
        x = inner + bvec
        t = 1.0 - 2.0 / (jnp.exp(x + x) + 1.0)
        for k in range(S // L):
            wu_v[i, pl.ds(16 * k, L)] = sv[k] + uv[k] * t
    pltpu.sync_copy(wu_v, ow_hbm.at[pl.ds(base, PB)])


def _fuse_body(wt_ref, ut_ref, wu_ref):
    wu_ref[...] = jnp.concatenate(
        [wt_ref[...].T, ut_ref[...].T], axis=-1)


def _s_body(st_lo_ref, st_hi_ref, s2_ref):
    s2_ref[:, 0:S] = st_lo_ref[...].T
    s2_ref[:, S:2 * S] = st_hi_ref[...].T


def _post_body(ow_ref, ot_ref):
    ot_ref[...] = ow_ref[:, 0:S].T


def kernel(m, s, w, b, u):
    wt = jnp.swapaxes(w, 0, 1)
    ut = jnp.swapaxes(u, 0, 1)
    st = jnp.swapaxes(s, 0, 1)
    wu = pl.pallas_call(
        _fuse_body,
        grid=(_TGRID,),
        in_specs=[
            pl.BlockSpec((S, _TBLK), lambda i: (0, i)),
            pl.BlockSpec((S, _TBLK), lambda i: (0, i)),
        ],
        out_specs=pl.BlockSpec((_TBLK, 2 * S), lambda i: (i, 0)),
        out_shape=jax.ShapeDtypeStruct((_TGRID * _TBLK, 2 * S), jnp.float32),
    )(wt, ut)
    s2 = pl.pallas_call(
        _s_body,
        grid=(B // 2 // _SBLK,),
        in_specs=[
            pl.BlockSpec((S, _SBLK), lambda i: (0, i)),
            pl.BlockSpec((S, _SBLK), lambda i: (0, i + B // 2 // _SBLK)),
        ],
        out_specs=pl.BlockSpec((_SBLK, 2 * S), lambda i: (i, 0)),
        out_shape=jax.ShapeDtypeStruct((B // 2, 2 * S), jnp.float32),
    )(st, st)

    mesh = plsc.VectorSubcoreMesh(core_axis_name="c", subcore_axis_name="s")
    run = pl.kernel(
        _sc_body,
        out_type=jax.ShapeDtypeStruct((B, 2 * S), jnp.float32),
        mesh=mesh,
        compiler_params=pltpu.CompilerParams(
            needs_layout_passes=False, use_tc_tiling_on_sc=False),
        scratch_types=[
            pltpu.VMEM((PB,), jnp.int32),
            pltpu.VMEM((PB, 2 * S), jnp.float32),
            pltpu.VMEM((PB, S), jnp.float32),
            pltpu.VMEM((PB + L,), jnp.float32),
            pltpu.SemaphoreType.DMA,
            pltpu.SemaphoreType.DMA,
            pltpu.SemaphoreType.DMA,
        ],
    )
    out_wide = run(m.astype(jnp.int32), s2, wu, b)
    out_t = pl.pallas_call(
        _post_body,
        grid=(B // _SBLK,),
        in_specs=[pl.BlockSpec((_SBLK, 2 * S), lambda i: (i, 0))],
        out_specs=pl.BlockSpec((S, _SBLK), lambda i: (0, i)),
        out_shape=jax.ShapeDtypeStruct((S, B), jnp.float32),
    )(out_wide)
    return jnp.swapaxes(out_t, 0, 1)


# speedup vs baseline: 1.0046x; 1.0046x over previous
import jax
import jax.numpy as jnp
from jax import lax
from jax.experimental import pallas as pl
from jax.experimental.pallas import tpu as pltpu
from jax.experimental.pallas import tpu_sc as plsc

B = 16384
S = 64
NC = 2
NS = 16
NW = NC * NS
L = 16
PB = B // NW
N_TABLE = 100000

_TBLK = 8192
_TGRID = (N_TABLE + _TBLK - 1) // _TBLK
_SBLK = 2048


def _sc_body(m_hbm, s2_hbm, wu_hbm, b_hbm, ow_hbm,
             idx_v, wu_v, s_v, bm_v, sem_wu, sem_b, sem_s):
    wid = lax.axis_index("s") * NC + lax.axis_index("c")
    base = wid * PB
    half = base // (B // 2)

    pltpu.sync_copy(m_hbm.at[pl.ds(base, PB)], idx_v)
    cwu = pltpu.make_async_copy(wu_hbm.at[idx_v], wu_v, sem_wu)
    cb = pltpu.make_async_copy(b_hbm.at[idx_v], bm_v.at[pl.ds(0, PB)], sem_b)
    cs = pltpu.make_async_copy(
        s2_hbm.at[pl.ds(base % (B // 2), PB), pl.ds(half * S, S)], s_v, sem_s)
    cwu.start()
    cb.start()
    cs.start()
    cwu.wait()
    cs.wait()
    cb.wait()

    lane15 = jnp.full((L,), 15, jnp.int32)
    lane0 = jnp.zeros((L,), jnp.int32)

    @plsc.parallel_loop(0, PB, 1, unroll=8)
    def _body(i):
        sv = [s_v[i, pl.ds(16 * k, L)] for k in range(S // L)]
        wv = [wu_v[i, pl.ds(16 * k, L)] for k in range(S // L)]
        uv = [wu_v[i, pl.ds(S + 16 * k, L)] for k in range(S // L)]
        p = (wv[0] * sv[0] + wv[1] * sv[1]) + (wv[2] * sv[2] + wv[3] * sv[3])
        c = plsc.cumsum(p)
        inner = jnp.take_along_axis(c, lane15, axis=0)
        bvec = jnp.take_along_axis(bm_v[pl.ds(i, L)], lane0, axis=0)
        x = inner + bvec
        t = 1.0 - 2.0 / (jnp.exp(x + x) + 1.0)
        for k in range(S // L):
            wu_v[i, pl.ds(16 * k, L)] = sv[k] + uv[k] * t
    pltpu.sync_copy(wu_v, ow_hbm.at[pl.ds(base, PB)])


def _fuse_body(wt_ref, ut_ref, wu_ref):
    wu_ref[...] = jnp.concatenate(
        [wt_ref[...].T, ut_ref[...].T], axis=-1)


def _s_body(st_lo_ref, st_hi_ref, s2_ref):
    s2_ref[:, 0:S] = st_lo_ref[...].T
    s2_ref[:, S:2 * S] = st_hi_ref[...].T


def _post_body(ow_ref, ot_ref):
    ot_ref[...] = ow_ref[:, 0:S].T


def kernel(m, s, w, b, u):
    wt = jnp.swapaxes(w, 0, 1)
    ut = jnp.swapaxes(u, 0, 1)
    st = jnp.swapaxes(s, 0, 1)
    wu = pl.pallas_call(
        _fuse_body,
        grid=(_TGRID,),
        in_specs=[
            pl.BlockSpec((S, _TBLK), lambda i: (0, i)),
            pl.BlockSpec((S, _TBLK), lambda i: (0, i)),
        ],
        out_specs=pl.BlockSpec((_TBLK, 2 * S), lambda i: (i, 0)),
        out_shape=jax.ShapeDtypeStruct((_TGRID * _TBLK, 2 * S), jnp.float32),
    )(wt, ut)
    s2 = pl.pallas_call(
        _s_body,
        grid=(B // 2 // _SBLK,),
        in_specs=[
            pl.BlockSpec((S, _SBLK), lambda i: (0, i)),
            pl.BlockSpec((S, _SBLK), lambda i: (0, i + B // 2 // _SBLK)),
        ],
        out_specs=pl.BlockSpec((_SBLK, 2 * S), lambda i: (i, 0)),
        out_shape=jax.ShapeDtypeStruct((B // 2, 2 * S), jnp.float32),
    )(st, st)

    mesh = plsc.VectorSubcoreMesh(core_axis_name="c", subcore_axis_name="s")
    run = pl.kernel(
        _sc_body,
        out_type=jax.ShapeDtypeStruct((B, 2 * S), jnp.float32),
        mesh=mesh,
        compiler_params=pltpu.CompilerParams(
            needs_layout_passes=False, use_tc_tiling_on_sc=False),
        scratch_types=[
            pltpu.VMEM((PB,), jnp.int32),
            pltpu.VMEM((PB, 2 * S), jnp.float32),
            pltpu.VMEM((PB, S), jnp.float32),
            pltpu.VMEM((PB + L,), jnp.float32),
            pltpu.SemaphoreType.DMA,
            pltpu.SemaphoreType.DMA,
            pltpu.SemaphoreType.DMA,
        ],
    )
    out_wide = run(m.astype(jnp.int32), s2, wu, b)
    out_t = pl.pallas_call(
        _post_body,
        grid=(B // _SBLK,),
        in_specs=[pl.BlockSpec((_SBLK, 2 * S), lambda i: (i, 0))],
        out_specs=pl.BlockSpec((S, _SBLK), lambda i: (0, i)),
        out_shape=jax.ShapeDtypeStruct((S, B), jnp.float32),
    )(out_wide)
    return jnp.swapaxes(out_t, 0, 1)
